# baseline (speedup 1.0000x reference)
import functools

import jax
import jax.numpy as jnp
from jax import lax
from jax.experimental import pallas as pl
from jax.experimental.pallas import tpu as pltpu
from jax.experimental.pallas import tpu_sc as plsc

_NB = 10
_RB = 16
_CAP = 700
_NBKT = 250


def _lane(vec, l):
    lv = jnp.broadcast_to(jnp.int32(l), (16,))
    return jnp.take_along_axis(
        vec, lv, axis=0, mode=lax.GatherScatterMode.PROMISE_IN_BOUNDS
    )[0]


def _make_gather(B: int, D: int):
    mesh = plsc.VectorSubcoreMesh(core_axis_name="c", subcore_axis_name="s")
    n_vec = B // 16

    @functools.partial(
        pl.kernel,
        mesh=mesh,
        out_type=jax.ShapeDtypeStruct((B * D,), jnp.float32),
        scratch_types=[
            pltpu.VMEM((B,), jnp.int32),
            pltpu.VMEM((256,), jnp.int32),
            pltpu.VMEM((16,), jnp.int32),
            pltpu.VMEM((16,), jnp.int32),
            pltpu.VMEM((_NB, D, 128), jnp.float32),
            pltpu.VMEM((_RB, D), jnp.float32),
            pltpu.VMEM((272,), jnp.int32),
            pltpu.SMEM((_CAP,), jnp.int32),
            pltpu.SMEM((_CAP,), jnp.int32),
            pltpu.SMEM((258,), jnp.int32),
            pltpu.SemaphoreType.DMA((_NB,)),
            pltpu.SemaphoreType.DMA((_RB,)),
        ],
        compiler_params=pltpu.CompilerParams(
            use_tc_tiling_on_sc=True, needs_layout_passes=False
        ),
    )
    def gather_kernel(
        idx_hbm, tab_t_hbm, out_hbm, idx_vm, counts_vm, stage_u, stage_p,
        blk_v, row_v, starts_vm, grp_u, grp_p, cursors, fsems, rsems,
    ):
        num_cores = lax.axis_size("c")
        wid = lax.axis_index("s") * num_cores + lax.axis_index("c")
        widv = jnp.broadcast_to(wid, (16,))
        zeros = jnp.zeros((16,), jnp.int32)
        ones = jnp.ones((16,), jnp.int32)
        iota = lax.iota(jnp.int32, 16)

        def fetch(b, t):
            tc = jnp.minimum(32 * b + wid, 7812)
            g = pl.multiple_of(tc << 7, 128)
            pltpu.async_copy(
                tab_t_hbm.at[:, pl.ds(g, 128)], blk_v.at[t], fsems.at[t]
            )

        for t in range(_NB):
            fetch(t, t)

        pltpu.sync_copy(idx_hbm, idx_vm)

        for z in range(16):
            counts_vm[pl.ds(16 * z, 16)] = zeros

        def hist_body(v, carry):
            for q in range(4):
                vec = idx_vm[pl.ds(pl.multiple_of(16 * (4 * v + q), 16), 16)]
                own = ((vec >> 7) & 31) == widv
                plsc.addupdate_scatter(
                    counts_vm.at[pl.ds(0, 256)], [vec >> 12], ones, mask=own
                )
            return carry

        lax.fori_loop(0, n_vec // 4, hist_body, 0)

        carry = jnp.int32(0)
        for z in range(16):
            c = counts_vm[pl.ds(16 * z, 16)]
            cum = plsc.cumsum(c)
            excl = (cum - c) + carry
            starts_vm[pl.ds(16 * z, 16)] = excl
            carry = carry + _lane(cum, 15)
        starts_vm[pl.ds(256, 16)] = jnp.broadcast_to(carry, (16,))
        total = carry

        def read_start(b):
            base = pl.multiple_of((b >> 4) << 4, 16)
            return _lane(starts_vm[pl.ds(base, 16)], b & 15)

        def reinit_cursors():
            def cinit(i, carry2):
                cursors[i] = read_start(i)
                return carry2

            lax.fori_loop(0, 258, cinit, 0)

        reinit_cursors()

        def place_body(v, carry2):
            for q in range(2):
                vec = idx_vm[pl.ds(pl.multiple_of(16 * (2 * v + q), 16), 16)]
                own = ((vec >> 7) & 31) == widv
                cnt = _lane(plsc.all_reduce_population_count(own), 0)

                @pl.when(cnt > 0)
                def _():
                    plsc.store_compressed(
                        stage_u.at[pl.ds(0, 16)], vec, mask=own
                    )
                    plsc.store_compressed(
                        stage_p.at[pl.ds(0, 16)],
                        iota + 16 * (2 * v + q),
                        mask=own,
                    )
                    su = stage_u[...]
                    sp = stage_p[...]

                    def put(k, carry3):
                        u = _lane(su, k)
                        b = u >> 12
                        o = cursors[b]
                        cursors[b] = o + 1

                        @pl.when(o < _CAP)
                        def _():
                            grp_u[o] = u
                            grp_p[o] = _lane(sp, k)

                        return carry3

                    lax.fori_loop(0, cnt, put, 0)

            return carry2

        lax.fori_loop(0, n_vec // 2, place_body, 0)

        def extract_row(blkref, u, r):
            cidx = jnp.broadcast_to(u & 127, (16,))
            for k in range(D // 16):
                ridx = lax.iota(jnp.int32, 16) + (16 * k)
                vals = plsc.load_gather(blkref, [ridx, cidx])
                row_v[r, pl.ds(16 * k, 16)] = vals

        def write_row(r, p):
            pltpu.async_copy(
                row_v.at[r], out_hbm.at[pl.ds(p * D, D)], rsems.at[r]
            )

        def drain_row(r):
            pltpu.make_async_copy(
                row_v.at[r], out_hbm.at[pl.ds(0, D)], rsems.at[r]
            ).wait()

        capped = jnp.minimum(total, _CAP)

        def bucket_outer(b0, carry2):
            for t in range(_NB):
                b = _NB * b0 + t
                pltpu.make_async_copy(
                    tab_t_hbm.at[:, pl.ds(0, 128)], blk_v.at[t], fsems.at[t]
                ).wait()
                s0 = jnp.minimum(read_start(b), _CAP)
                s1 = jnp.minimum(read_start(b + 1), _CAP)

                def emit(e, carry3):
                    r = e & (_RB - 1)

                    @pl.when(e >= _RB)
                    def _():
                        drain_row(r)

                    u = grp_u[e]
                    extract_row(blk_v.at[t], u, r)
                    write_row(r, grp_p[e])
                    return carry3

                lax.fori_loop(s0, s1, emit, 0)

                @pl.when(b + _NB < _NBKT)
                def _():
                    fetch(b + _NB, t)

            return carry2

        lax.fori_loop(0, _NBKT // _NB, bucket_outer, 0)

        def final_drain(j, carry2):
            drain_row(j & (_RB - 1))
            return carry2

        lax.fori_loop(0, jnp.minimum(capped, _RB), final_drain, 0)

        @pl.when(total > _CAP)
        def _():
            reinit_cursors()

            def scan_body(v, carry2):
                vec = idx_vm[pl.ds(pl.multiple_of(16 * v, 16), 16)]
                own = ((vec >> 7) & 31) == widv
                cnt = _lane(plsc.all_reduce_population_count(own), 0)

                @pl.when(cnt > 0)
                def _():
                    plsc.store_compressed(stage_u.at[pl.ds(0, 16)], vec, mask=own)
                    plsc.store_compressed(stage_p.at[pl.ds(0, 16)], iota + 16 * v, mask=own)
                    su = stage_u[...]
                    sp = stage_p[...]

                    def put(k, carry3):
                        u = _lane(su, k)
                        b = u >> 12
                        o = cursors[b]
                        cursors[b] = o + 1

                        @pl.when(o >= _CAP)
                        def _():
                            g = pl.multiple_of((u >> 7) << 7, 128)
                            pltpu.async_copy(
                                tab_t_hbm.at[:, pl.ds(g, 128)],
                                blk_v.at[0],
                                fsems.at[0],
                            ).wait()
                            extract_row(blk_v.at[0], u, 0)
                            write_row(0, _lane(sp, k))
                            drain_row(0)

                        return carry3

                    lax.fori_loop(0, cnt, put, 0)

                return carry2

            lax.fori_loop(0, n_vec, scan_body, 0)

    return gather_kernel


def kernel(UserIdx, embed_user_MLP):
    (B,) = UserIdx.shape
    V, D = embed_user_MLP.shape
    idx = UserIdx.astype(jnp.int32)
    tab_t = embed_user_MLP.T
    fn = _make_gather(B, D)
    out = fn(idx, tab_t)
    return out.reshape(B, D)


# speedup vs baseline: 1.0508x; 1.0508x over previous
import functools

import jax
import jax.numpy as jnp
from jax import lax
from jax.experimental import pallas as pl
from jax.experimental.pallas import tpu as pltpu
from jax.experimental.pallas import tpu_sc as plsc

_NB = 8
_RB = 16
_CAP = 700
_NBKT = 248


def _lane(vec, l):
    lv = jnp.broadcast_to(jnp.int32(l), (16,))
    return jnp.take_along_axis(
        vec, lv, axis=0, mode=lax.GatherScatterMode.PROMISE_IN_BOUNDS
    )[0]


def _make_gather(B: int, D: int):
    mesh = plsc.VectorSubcoreMesh(core_axis_name="c", subcore_axis_name="s")
    n_vec = B // 16

    @functools.partial(
        pl.kernel,
        mesh=mesh,
        out_type=jax.ShapeDtypeStruct((B * D,), jnp.float32),
        scratch_types=[
            pltpu.VMEM((B,), jnp.int32),
            pltpu.VMEM((256,), jnp.int32),
            pltpu.VMEM((16,), jnp.int32),
            pltpu.VMEM((16,), jnp.int32),
            pltpu.VMEM((_NB, D, 128), jnp.float32),
            pltpu.VMEM((_RB, D), jnp.float32),
            pltpu.VMEM((272,), jnp.int32),
            pltpu.SMEM((_CAP,), jnp.int32),
            pltpu.SMEM((_CAP,), jnp.int32),
            pltpu.SMEM((258,), jnp.int32),
            pltpu.SemaphoreType.DMA((_NB,)),
            pltpu.SemaphoreType.DMA((_RB,)),
        ],
        compiler_params=pltpu.CompilerParams(
            use_tc_tiling_on_sc=True, needs_layout_passes=False
        ),
    )
    def gather_kernel(
        idx_hbm, tab_t_hbm, out_hbm, idx_vm, counts_vm, stage_u, stage_p,
        blk_v, row_v, starts_vm, grp_u, grp_p, cursors, fsems, rsems,
    ):
        num_cores = lax.axis_size("c")
        wid = lax.axis_index("s") * num_cores + lax.axis_index("c")
        widv = jnp.broadcast_to(wid, (16,))
        zeros = jnp.zeros((16,), jnp.int32)
        ones = jnp.ones((16,), jnp.int32)
        iota = lax.iota(jnp.int32, 16)

        def fetch(b, t):
            tc = jnp.minimum(32 * b + wid, 7812)
            g = pl.multiple_of(tc << 7, 128)
            pltpu.async_copy(
                tab_t_hbm.at[:, pl.ds(g, 128)], blk_v.at[t], fsems.at[t]
            )

        for t in range(_NB):
            fetch(t, t)

        pltpu.sync_copy(idx_hbm, idx_vm)

        for z in range(16):
            counts_vm[pl.ds(16 * z, 16)] = zeros

        def hist_body(v, carry):
            for q in range(4):
                vec = idx_vm[pl.ds(pl.multiple_of(16 * (4 * v + q), 16), 16)]
                own = ((vec >> 7) & 31) == widv
                plsc.addupdate_scatter(
                    counts_vm.at[pl.ds(0, 256)], [vec >> 12], ones, mask=own
                )
            return carry

        lax.fori_loop(0, n_vec // 4, hist_body, 0)

        carry = jnp.int32(0)
        for z in range(16):
            c = counts_vm[pl.ds(16 * z, 16)]
            cum = plsc.cumsum(c)
            excl = (cum - c) + carry
            starts_vm[pl.ds(16 * z, 16)] = excl
            carry = carry + _lane(cum, 15)
        starts_vm[pl.ds(256, 16)] = jnp.broadcast_to(carry, (16,))
        total = carry

        def read_start(b):
            base = pl.multiple_of((b >> 4) << 4, 16)
            return _lane(starts_vm[pl.ds(base, 16)], b & 15)

        def reinit_cursors():
            def cinit(i, carry2):
                cursors[i] = read_start(i)
                return carry2

            lax.fori_loop(0, 258, cinit, 0)

        reinit_cursors()

        def place_body(v, carry2):
            for q in range(2):
                vec = idx_vm[pl.ds(pl.multiple_of(16 * (2 * v + q), 16), 16)]
                own = ((vec >> 7) & 31) == widv
                cnt = _lane(plsc.all_reduce_population_count(own), 0)

                @pl.when(cnt > 0)
                def _():
                    plsc.store_compressed(
                        stage_u.at[pl.ds(0, 16)], vec, mask=own
                    )
                    plsc.store_compressed(
                        stage_p.at[pl.ds(0, 16)],
                        iota + 16 * (2 * v + q),
                        mask=own,
                    )
                    su = stage_u[...]
                    sp = stage_p[...]

                    def put(k, carry3):
                        u = _lane(su, k)
                        b = u >> 12
                        o = cursors[b]
                        cursors[b] = o + 1

                        @pl.when(o < _CAP)
                        def _():
                            grp_u[o] = u
                            grp_p[o] = _lane(sp, k)

                        return carry3

                    lax.fori_loop(0, cnt, put, 0)

            return carry2

        lax.fori_loop(0, n_vec // 2, place_body, 0)

        def extract_row(blkref, u, r):
            cidx = jnp.broadcast_to(u & 127, (16,))
            for k in range(D // 16):
                ridx = lax.iota(jnp.int32, 16) + (16 * k)
                vals = plsc.load_gather(blkref, [ridx, cidx])
                row_v[r, pl.ds(16 * k, 16)] = vals

        def write_row(r, p):
            pltpu.async_copy(
                row_v.at[r], out_hbm.at[pl.ds(p * D, D)], rsems.at[r]
            )

        def drain_row(r):
            pltpu.make_async_copy(
                row_v.at[r], out_hbm.at[pl.ds(0, D)], rsems.at[r]
            ).wait()

        capped = jnp.minimum(total, _CAP)

        def bucket_outer(b0, carry2):
            for t in range(_NB):
                b = _NB * b0 + t
                pltpu.make_async_copy(
                    tab_t_hbm.at[:, pl.ds(0, 128)], blk_v.at[t], fsems.at[t]
                ).wait()
                s0 = jnp.minimum(read_start(b), _CAP)
                s1 = jnp.minimum(read_start(b + 1), _CAP)

                def emit(e, carry3):
                    r = e & (_RB - 1)

                    @pl.when(e >= _RB)
                    def _():
                        drain_row(r)

                    u = grp_u[e]
                    extract_row(blk_v.at[t], u, r)
                    write_row(r, grp_p[e])
                    return carry3

                lax.fori_loop(s0, s1, emit, 0)

                @pl.when(b + _NB < _NBKT)
                def _():
                    fetch(b + _NB, t)

            return carry2

        lax.fori_loop(0, _NBKT // _NB, bucket_outer, 0)

        def final_drain(j, carry2):
            drain_row(j & (_RB - 1))
            return carry2

        lax.fori_loop(0, jnp.minimum(capped, _RB), final_drain, 0)

        @pl.when(total > _CAP)
        def _():
            reinit_cursors()

            def scan_body(v, carry2):
                vec = idx_vm[pl.ds(pl.multiple_of(16 * v, 16), 16)]
                own = ((vec >> 7) & 31) == widv
                cnt = _lane(plsc.all_reduce_population_count(own), 0)

                @pl.when(cnt > 0)
                def _():
                    plsc.store_compressed(stage_u.at[pl.ds(0, 16)], vec, mask=own)
                    plsc.store_compressed(stage_p.at[pl.ds(0, 16)], iota + 16 * v, mask=own)
                    su = stage_u[...]
                    sp = stage_p[...]

                    def put(k, carry3):
                        u = _lane(su, k)
                        b = u >> 12
                        o = cursors[b]
                        cursors[b] = o + 1

                        @pl.when(o >= _CAP)
                        def _():
                            g = pl.multiple_of((u >> 7) << 7, 128)
                            pltpu.async_copy(
                                tab_t_hbm.at[:, pl.ds(g, 128)],
                                blk_v.at[0],
                                fsems.at[0],
                            ).wait()
                            extract_row(blk_v.at[0], u, 0)
                            write_row(0, _lane(sp, k))
                            drain_row(0)

                        return carry3

                    lax.fori_loop(0, cnt, put, 0)

                return carry2

            lax.fori_loop(0, n_vec, scan_body, 0)

    return gather_kernel


def kernel(UserIdx, embed_user_MLP):
    (B,) = UserIdx.shape
    V, D = embed_user_MLP.shape
    idx = UserIdx.astype(jnp.int32)
    tab_t = embed_user_MLP.T
    fn = _make_gather(B, D)
    out = fn(idx, tab_t)
    return out.reshape(B, D)
